# baseline (speedup 1.0000x reference)
import functools

import jax
import jax.numpy as jnp
from jax.experimental import pallas as pl
from jax.experimental.pallas import tpu as pltpu

BL = 512
NBUF = 4


def _body(n_ref, bagf_ref, blkf_ref, bagc_ref, blkc_ref, tot_ref,
          x_ref, w1_ref, b1_ref, w2_ref, b2_ref, out_ref,
          xbuf, acc_ref, sems, *, bl: int):
    groups = tot_ref[0]

    def issue(t, slot):
        b = bagf_ref[t]
        jj = blkf_ref[t]
        pltpu.make_async_copy(
            x_ref.at[b, pl.ds(jj * bl, bl), :], xbuf.at[slot],
            sems.at[slot]).start()

    def wait(t, slot):
        b = bagf_ref[t]
        jj = blkf_ref[t]
        pltpu.make_async_copy(
            x_ref.at[b, pl.ds(jj * bl, bl), :], xbuf.at[slot],
            sems.at[slot]).wait()

    for t0 in range(NBUF - 1):
        issue(t0, t0)

    ones8 = jnp.ones((8, bl), jnp.float32)

    def one_step(t, slot):
        b = bagc_ref[t]
        jj = blkc_ref[t]
        nb = n_ref[b]

        wait(t, slot)
        issue(t + NBUF - 1, (slot + NBUF - 1) % NBUF)

        @pl.when(jj == 0)
        def _init():
            acc_ref[...] = jnp.zeros_like(acc_ref)

        def hidden():
            h = jnp.dot(xbuf[slot], w1_ref[...],
                        preferred_element_type=jnp.float32)
            return jnp.maximum(h + b1_ref[...], 0.0)

        @pl.when((jj + 1) * bl <= nb)
        def _compute_full():
            h = hidden()
            acc_ref[...] += jnp.dot(ones8, h,
                                    preferred_element_type=jnp.float32)

        @pl.when((jj * bl < nb) & ((jj + 1) * bl > nb))
        def _compute_partial():
            h = hidden()
            rows = jax.lax.broadcasted_iota(jnp.int32, (bl, 1), 0) + jj * bl
            h = jnp.where(rows < nb, h, 0.0)
            acc_ref[...] += jnp.dot(ones8, h,
                                    preferred_element_type=jnp.float32)

        @pl.when((jj + 1) * bl >= nb)
        def _finalize():
            pooled = acc_ref[0:1] / nb.astype(jnp.float32)
            res = jnp.dot(pooled, w2_ref[...],
                          preferred_element_type=jnp.float32) + b2_ref[...]
            out_ref[b] = res

    def step(g, carry):
        t = NBUF * g
        for k in range(NBUF):
            one_step(t + k, k)
        return carry

    jax.lax.fori_loop(0, groups, step, 0)

    for k in range(1, NBUF):
        t = NBUF * groups - 1 + k
        wait(t, jax.lax.rem(t, NBUF))


def kernel(x, n_instances, W1, b1, W2, b2):
    B, L, D = x.shape
    DO = W2.shape[1]
    nj = L // BL
    n32 = n_instances.astype(jnp.int32)

    nblk = (n32 + BL - 1) // BL
    ends = jnp.cumsum(nblk)
    starts = ends - nblk
    total = ends[-1]
    groups = (total + NBUF - 1) // NBUF
    tpad = B * nj + 2 * NBUF + 4
    t_idx = jnp.arange(tpad, dtype=jnp.int32)
    bag_raw = jnp.searchsorted(ends, t_idx, side="right").astype(jnp.int32)
    valid = t_idx < total
    bagf = jnp.where(valid, jnp.minimum(bag_raw, B - 1), 0)
    blkf = jnp.where(valid, t_idx - starts[bagf], 0)
    bagc = jnp.where(valid, bagf, B)
    blkc = jnp.where(valid, blkf, 1)
    n_pad = jnp.concatenate([n32, jnp.ones((1,), jnp.int32)])

    grid_spec = pltpu.PrefetchScalarGridSpec(
        num_scalar_prefetch=6,
        grid=(1,),
        in_specs=[
            pl.BlockSpec(memory_space=pl.ANY),
            pl.BlockSpec((D, D), lambda i, *_: (0, 0)),
            pl.BlockSpec((1, D), lambda i, *_: (0, 0)),
            pl.BlockSpec((D, DO), lambda i, *_: (0, 0)),
            pl.BlockSpec((1, DO), lambda i, *_: (0, 0)),
        ],
        out_specs=pl.BlockSpec((B + 1, 1, DO), lambda i, *_: (0, 0, 0)),
        scratch_shapes=[
            pltpu.VMEM((NBUF, BL, D), jnp.float32),
            pltpu.VMEM((8, D), jnp.float32),
            pltpu.SemaphoreType.DMA((NBUF,)),
        ],
    )

    out = pl.pallas_call(
        functools.partial(_body, bl=BL),
        grid_spec=grid_spec,
        out_shape=jax.ShapeDtypeStruct((B + 1, 1, DO), jnp.float32),
    )(n_pad, bagf, blkf, bagc, blkc, groups.reshape(1), x, W1,
      b1.reshape(1, D), W2, b2.reshape(1, DO))
    return out[:B].reshape(B, DO)


# speedup vs baseline: 1.2273x; 1.2273x over previous
import functools

import jax
import jax.numpy as jnp
from jax.experimental import pallas as pl
from jax.experimental.pallas import tpu as pltpu

BL = 512
NBUF = 4


def _body(n_ref, bag_ref, blk_ref, tot_ref, x_ref, w1_ref, b1_ref, w2_ref,
          b2_ref, out_ref, xbuf, acc_ref, sems, *, bl: int):
    total = tot_ref[0]

    def issue(t):
        slot = jax.lax.rem(t, NBUF)
        b = bag_ref[t]
        jj = blk_ref[t]
        pltpu.make_async_copy(
            x_ref.at[b, pl.ds(jj * bl, bl), :], xbuf.at[slot],
            sems.at[slot]).start()

    for t0 in range(NBUF - 1):
        @pl.when(t0 < total)
        def _prologue():
            issue(t0)

    def step(t, carry):
        slot = jax.lax.rem(t, NBUF)
        b = bag_ref[t]
        jj = blk_ref[t]
        nb = n_ref[b]

        pltpu.make_async_copy(
            x_ref.at[b, pl.ds(jj * bl, bl), :], xbuf.at[slot],
            sems.at[slot]).wait()

        @pl.when(t + NBUF - 1 < total)
        def _issue_ahead():
            issue(t + NBUF - 1)

        @pl.when(jj == 0)
        def _init():
            acc_ref[...] = jnp.zeros_like(acc_ref)

        def hidden():
            h = jnp.dot(xbuf[slot], w1_ref[...],
                        preferred_element_type=jnp.float32)
            return jnp.maximum(h + b1_ref[...], 0.0)

        @pl.when((jj + 1) * bl <= nb)
        def _compute_full():
            h = hidden()
            acc_ref[...] += jnp.sum(h.reshape(bl // 8, 8, -1), axis=0)

        @pl.when((jj + 1) * bl > nb)
        def _compute_partial():
            h = hidden()
            rows = jax.lax.broadcasted_iota(jnp.int32, (bl, 1), 0) + jj * bl
            h = jnp.where(rows < nb, h, 0.0)
            acc_ref[...] += jnp.sum(h.reshape(bl // 8, 8, -1), axis=0)

        @pl.when((jj + 1) * bl >= nb)
        def _finalize():
            pooled = jnp.sum(acc_ref[...], axis=0, keepdims=True)
            pooled = pooled / nb.astype(jnp.float32)
            res = jnp.dot(pooled, w2_ref[...],
                          preferred_element_type=jnp.float32) + b2_ref[...]
            out_ref[b] = res

        return carry

    jax.lax.fori_loop(0, total, step, 0)


def kernel(x, n_instances, W1, b1, W2, b2):
    B, L, D = x.shape
    DO = W2.shape[1]
    nj = L // BL
    n32 = n_instances.astype(jnp.int32)

    nblk = (n32 + BL - 1) // BL
    ends = jnp.cumsum(nblk)
    starts = ends - nblk
    total = ends[-1:]
    t_idx = jnp.arange(B * nj, dtype=jnp.int32)
    bag_tbl = jnp.minimum(
        jnp.searchsorted(ends, t_idx, side="right").astype(jnp.int32), B - 1)
    blk_tbl = t_idx - starts[bag_tbl]

    grid_spec = pltpu.PrefetchScalarGridSpec(
        num_scalar_prefetch=4,
        grid=(1,),
        in_specs=[
            pl.BlockSpec(memory_space=pl.ANY),
            pl.BlockSpec((D, D), lambda i, *_: (0, 0)),
            pl.BlockSpec((1, D), lambda i, *_: (0, 0)),
            pl.BlockSpec((D, DO), lambda i, *_: (0, 0)),
            pl.BlockSpec((1, DO), lambda i, *_: (0, 0)),
        ],
        out_specs=pl.BlockSpec((B, 1, DO), lambda i, *_: (0, 0, 0)),
        scratch_shapes=[
            pltpu.VMEM((NBUF, BL, D), jnp.float32),
            pltpu.VMEM((8, D), jnp.float32),
            pltpu.SemaphoreType.DMA((NBUF,)),
        ],
    )

    out = pl.pallas_call(
        functools.partial(_body, bl=BL),
        grid_spec=grid_spec,
        out_shape=jax.ShapeDtypeStruct((B, 1, DO), jnp.float32),
    )(n32, bag_tbl, blk_tbl, total, x, W1,
      b1.reshape(1, D), W2, b2.reshape(1, DO))
    return out.reshape(B, DO)
